# baseline (speedup 1.0000x reference)
import functools

import jax
import jax.numpy as jnp
from jax import lax
from jax.experimental import pallas as pl
from jax.experimental.pallas import tpu as pltpu
from jax.experimental.pallas import tpu_sc as plsc

BATCH = 16384
NF = 64
HP = 16
NH = 10
NP = 5

_info = plsc.get_sparse_core_info()
_NC, _NS = _info.num_cores, _info.num_subcores
_NW = _NC * _NS
_BPW = BATCH // _NW


def _project_body(lo, w_ref, t_ref, *out_refs):
    wh = w_ref[:, :NF] if lo else w_ref[:, NF:]
    p = jnp.dot(wh, t_ref[:], preferred_element_type=jnp.float32)
    pb = lax.bitcast_convert_type(p.astype(jnp.bfloat16), jnp.uint16).astype(jnp.uint32)
    for k, o_ref in enumerate(out_refs):
        o_ref[:] = pb[2 * k, :] | (pb[2 * k + 1, :] << 16)


def _project(w1, table_t, cols, lo):
    n = table_t.shape[1]
    grid = pl.cdiv(n, cols)
    return pl.pallas_call(
        functools.partial(_project_body, lo),
        grid=(grid,),
        in_specs=[
            pl.BlockSpec((NH, 2 * NF), lambda i: (0, 0)),
            pl.BlockSpec((NF, cols), lambda i: (0, i)),
        ],
        out_specs=[pl.BlockSpec((cols,), lambda i: (i,)) for _ in range(NP)],
        out_shape=[jax.ShapeDtypeStruct((n,), jnp.uint32) for _ in range(NP)],
    )(w1, table_t)


def _mgather_body(movies_hbm, *rest):
    pm_hbm = rest[:NP]
    out_hbm = rest[NP:2 * NP]
    idx_m, gath, sem = rest[2 * NP:2 * NP + 3]

    wid = lax.axis_index("s") * _NC + lax.axis_index("c")
    base = wid * _BPW
    pltpu.sync_copy(movies_hbm.at[pl.ds(base, _BPW)], idx_m)
    copies = [pltpu.async_copy(pm_hbm[f].at[idx_m], gath.at[f], sem)
              for f in range(NP)]
    for c in copies:
        c.wait()
    for f in range(NP):
        pltpu.sync_copy(gath.at[f], out_hbm[f].at[pl.ds(base, _BPW)])


_sc_mgather = functools.partial(
    pl.kernel,
    out_type=[jax.ShapeDtypeStruct((BATCH,), jnp.uint32) for _ in range(NP)],
    mesh=plsc.VectorSubcoreMesh(core_axis_name="c", subcore_axis_name="s"),
    compiler_params=pltpu.CompilerParams(use_tc_tiling_on_sc=False),
    scratch_types=[
        pltpu.VMEM((_BPW,), jnp.int32),
        pltpu.VMEM((NP, _BPW), jnp.uint32),
        pltpu.SemaphoreType.DMA,
    ],
)(_mgather_body)


def _tail_body(users_hbm, w2_hbm, b1_hbm, b2_hbm, *rest):
    pu_hbm = rest[:NP]
    hm_hbm = rest[NP:2 * NP]
    out_hbm = rest[2 * NP]
    idx_u, coef_v, acc_v, gath = rest[2 * NP + 1:2 * NP + 5]
    sem = rest[2 * NP + 5]

    wid = lax.axis_index("s") * _NC + lax.axis_index("c")
    base = wid * _BPW
    pltpu.sync_copy(users_hbm.at[pl.ds(base, _BPW)], idx_u)
    pltpu.sync_copy(w2_hbm, coef_v.at[0])
    pltpu.sync_copy(b1_hbm, coef_v.at[1])
    pltpu.sync_copy(b2_hbm, coef_v.at[2, pl.ds(0, 1)])

    copies = [pltpu.async_copy(pu_hbm[f].at[idx_u], gath.at[0, f], sem)
              for f in range(NP)]
    copies += [pltpu.async_copy(hm_hbm[f].at[pl.ds(base, _BPW)], gath.at[1, f], sem)
               for f in range(NP)]
    for c in copies:
        c.wait()

    w2v = coef_v[0]
    b1v = coef_v[1]
    b2v = coef_v[2]

    def group(g, carry):
        sl = pl.ds(g * 16, 16)
        acc = jnp.zeros((16,), jnp.float32) + b2v[0]
        for k in range(NP):
            wu = gath[0, k, sl]
            wm = gath[1, k, sl]
            for half in range(2):
                if half == 0:
                    pu_f = plsc.bitcast(wu << 16, jnp.float32)
                    pm_f = plsc.bitcast(wm << 16, jnp.float32)
                else:
                    pu_f = plsc.bitcast(wu & jnp.uint32(0xFFFF0000), jnp.float32)
                    pm_f = plsc.bitcast(wm & jnp.uint32(0xFFFF0000), jnp.float32)
                f = 2 * k + half
                h = pu_f + pm_f + b1v[f]
                acc = acc + w2v[f] * jnp.maximum(h, 0.0)
        sig = 1.0 / (1.0 + jnp.exp(-acc))
        acc_v[sl] = sig * 6.0 - 0.5
        return carry

    lax.fori_loop(0, _BPW // 16, group, 0)
    pltpu.sync_copy(acc_v, out_hbm.at[pl.ds(base, _BPW)])


_sc_tail = functools.partial(
    pl.kernel,
    out_type=jax.ShapeDtypeStruct((BATCH,), jnp.float32),
    mesh=plsc.VectorSubcoreMesh(core_axis_name="c", subcore_axis_name="s"),
    compiler_params=pltpu.CompilerParams(use_tc_tiling_on_sc=False,
                                         needs_layout_passes=False),
    scratch_types=[
        pltpu.VMEM((_BPW,), jnp.int32),
        pltpu.VMEM((3, HP), jnp.float32),
        pltpu.VMEM((_BPW,), jnp.float32),
        pltpu.VMEM((2, NP, _BPW), jnp.uint32),
        pltpu.SemaphoreType.DMA,
    ],
)(_tail_body)


def kernel(users, movies, U, M, W1, b1, W2, b2):
    users = users.astype(jnp.int32)
    movies = movies.astype(jnp.int32)
    w2p = jnp.zeros((HP,), jnp.float32).at[:10].set(W2[0])
    b1p = jnp.zeros((HP,), jnp.float32).at[:10].set(b1)
    pm = _project(W1, M.T, 32768, False)
    hm = _sc_mgather(movies, *pm)
    pu = _project(W1, U.T, 65536, True)
    out = _sc_tail(users, w2p, b1p, b2, *pu, *hm)
    return out
